# initial kernel scaffold (unmeasured)
import jax
import jax.numpy as jnp
from jax import lax
from jax.experimental import pallas as pl
from jax.experimental.pallas import tpu as pltpu

N_DEV = 4


def kernel(x, w_mat):
    m, k_shard = x.shape
    _, n = w_mat.shape
    m_chunk = m // N_DEV

    x_bf = x.astype(jnp.bfloat16)
    w_bf = w_mat.astype(jnp.bfloat16)

    def body(x_ref, w_ref, out_ref, send_buf, recv_buf, send_sems, recv_sems,
             credit_sem):
        me = lax.axis_index("i")
        right = lax.rem(me + 1, N_DEV)
        left = lax.rem(me + N_DEV - 1, N_DEV)

        barrier_sem = pltpu.get_barrier_semaphore()
        for nbr in (left, right):
            pl.semaphore_signal(barrier_sem, inc=1, device_id=(nbr,),
                                device_id_type=pl.DeviceIdType.MESH)
        pl.semaphore_wait(barrier_sem, 2)

        def mod(v):
            return lax.rem(v + 2 * N_DEV, N_DEV)

        def local_partial(c):
            xs = x_ref[pl.ds(c * m_chunk, m_chunk), :]
            return jnp.dot(xs, w_ref[:, :], preferred_element_type=jnp.float32)

        def hop(h, src, dst):
            return pltpu.make_async_remote_copy(
                src_ref=src, dst_ref=dst,
                send_sem=send_sems.at[h], recv_sem=recv_sems.at[h],
                device_id=(right,), device_id_type=pl.DeviceIdType.MESH,
            )

        def credit_to_left():
            pl.semaphore_signal(credit_sem, inc=1, device_id=(left,),
                                device_id_type=pl.DeviceIdType.MESH)

        send_buf[0, :, :] = local_partial(me).astype(jnp.bfloat16)
        acc = None
        for h in range(N_DEV - 1):
            slot = h % 2
            rdma = hop(h, send_buf.at[slot], recv_buf.at[slot])
            if h >= 2:
                pl.semaphore_wait(credit_sem, 1)
            rdma.start()
            nxt = local_partial(mod(me - h - 1))
            rdma.wait()
            acc = nxt + recv_buf[slot, :, :].astype(jnp.float32)
            if h < N_DEV - 2:
                send_buf[1 - slot, :, :] = acc.astype(jnp.bfloat16)
                credit_to_left()

        own = mod(me + 1)
        full_bf = jnp.maximum(acc, 0.0).astype(jnp.bfloat16)
        out_ref[pl.ds(own * m_chunk, m_chunk), :] = full_bf
        send_buf[1, :, :] = full_bf
        credit_to_left()

        r3 = hop(3, send_buf.at[1], recv_buf.at[1])
        pl.semaphore_wait(credit_sem, 1)
        r3.start()
        r3.wait()
        out_ref[pl.ds(me * m_chunk, m_chunk), :] = recv_buf[1, :, :]

        r4 = hop(4, recv_buf.at[1], recv_buf.at[0])
        pl.semaphore_wait(credit_sem, 1)
        r4.start()
        r4.wait()
        credit_to_left()
        out_ref[pl.ds(mod(me - 1) * m_chunk, m_chunk), :] = recv_buf[0, :, :]

        r5 = hop(5, recv_buf.at[0], recv_buf.at[1])
        pl.semaphore_wait(credit_sem, 1)
        r5.start()
        r5.wait()
        out_ref[pl.ds(mod(me - 2) * m_chunk, m_chunk), :] = recv_buf[1, :, :]

    return pl.pallas_call(
        body,
        out_shape=jax.ShapeDtypeStruct((m, n), jnp.bfloat16),
        in_specs=[pl.BlockSpec(memory_space=pltpu.VMEM),
                  pl.BlockSpec(memory_space=pltpu.VMEM)],
        out_specs=pl.BlockSpec(memory_space=pltpu.VMEM),
        scratch_shapes=[
            pltpu.VMEM((2, m_chunk, n), jnp.bfloat16),
            pltpu.VMEM((2, m_chunk, n), jnp.bfloat16),
            pltpu.SemaphoreType.DMA((6,)),
            pltpu.SemaphoreType.DMA((6,)),
            pltpu.SemaphoreType.REGULAR,
        ],
        compiler_params=pltpu.CompilerParams(collective_id=0),
    )(x_bf, w_bf)


# baseline (device time: 321476 ns/iter reference)
import jax
import jax.numpy as jnp
from jax import lax
from jax.experimental import pallas as pl
from jax.experimental.pallas import tpu as pltpu

N_DEV = 4
SUB = 2


def kernel(x, w_mat):
    m, k_shard = x.shape
    _, n = w_mat.shape
    m_chunk = m // N_DEV
    m_sub = m_chunk // SUB

    x_bf = x.astype(jnp.bfloat16)
    w_bf = w_mat.astype(jnp.bfloat16)

    def body(x_ref, w_ref, out_ref, send_buf, recv_buf, acc_buf,
             send_sems, recv_sems, credit_sem):
        me = lax.axis_index("i")
        right = lax.rem(me + 1, N_DEV)
        left = lax.rem(me + N_DEV - 1, N_DEV)

        barrier_sem = pltpu.get_barrier_semaphore()
        for nbr in (left, right):
            pl.semaphore_signal(barrier_sem, inc=1, device_id=(nbr,),
                                device_id_type=pl.DeviceIdType.MESH)
        pl.semaphore_wait(barrier_sem, 2)

        def mod(v):
            return lax.rem(v + 2 * N_DEV, N_DEV)

        def partial_sub(c, s):
            xs = x_ref[pl.ds(c * m_chunk + s * m_sub, m_sub), :]
            return jnp.dot(xs, w_ref[:, :], preferred_element_type=jnp.float32)

        def partial_into_acc(c):
            for s in range(SUB):
                acc_buf[pl.ds(s * m_sub, m_sub), :] = partial_sub(c, s)

        def hop(h, src, dst):
            return pltpu.make_async_remote_copy(
                src_ref=src, dst_ref=dst,
                send_sem=send_sems.at[h], recv_sem=recv_sems.at[h],
                device_id=(right,), device_id_type=pl.DeviceIdType.MESH,
            )

        def credit_to_left():
            pl.semaphore_signal(credit_sem, inc=1, device_id=(left,),
                                device_id_type=pl.DeviceIdType.MESH)

        for s in range(SUB):
            send_buf[0, pl.ds(s * m_sub, m_sub), :] = (
                partial_sub(me, s).astype(jnp.bfloat16))
        for h in range(N_DEV - 1):
            slot = h % 2
            rdma = hop(h, send_buf.at[slot], recv_buf.at[slot])
            if h >= 2:
                pl.semaphore_wait(credit_sem, 1)
            rdma.start()
            partial_into_acc(mod(me - h - 1))
            rdma.wait()
            for s in range(SUB):
                rows = pl.ds(s * m_sub, m_sub)
                acc_buf[rows, :] = (
                    acc_buf[rows, :]
                    + recv_buf[slot, rows, :].astype(jnp.float32))
            if h < N_DEV - 2:
                for s in range(SUB):
                    rows = pl.ds(s * m_sub, m_sub)
                    send_buf[1 - slot, rows, :] = (
                        acc_buf[rows, :].astype(jnp.bfloat16))
                credit_to_left()

        own = mod(me + 1)
        for s in range(SUB):
            rows = pl.ds(s * m_sub, m_sub)
            full_bf = jnp.maximum(acc_buf[rows, :], 0.0).astype(jnp.bfloat16)
            out_ref[pl.ds(own * m_chunk + s * m_sub, m_sub), :] = full_bf
            send_buf[1, rows, :] = full_bf
        credit_to_left()

        r3 = hop(3, send_buf.at[1], recv_buf.at[1])
        pl.semaphore_wait(credit_sem, 1)
        r3.start()
        r3.wait()
        out_ref[pl.ds(me * m_chunk, m_chunk), :] = recv_buf[1, :, :]

        r4 = hop(4, recv_buf.at[1], recv_buf.at[0])
        pl.semaphore_wait(credit_sem, 1)
        r4.start()
        r4.wait()
        credit_to_left()
        out_ref[pl.ds(mod(me - 1) * m_chunk, m_chunk), :] = recv_buf[0, :, :]

        r5 = hop(5, recv_buf.at[0], recv_buf.at[1])
        pl.semaphore_wait(credit_sem, 1)
        r5.start()
        r5.wait()
        out_ref[pl.ds(mod(me - 2) * m_chunk, m_chunk), :] = recv_buf[1, :, :]

    return pl.pallas_call(
        body,
        out_shape=jax.ShapeDtypeStruct((m, n), jnp.bfloat16),
        in_specs=[pl.BlockSpec(memory_space=pltpu.VMEM),
                  pl.BlockSpec(memory_space=pltpu.VMEM)],
        out_specs=pl.BlockSpec(memory_space=pltpu.VMEM),
        scratch_shapes=[
            pltpu.VMEM((2, m_chunk, n), jnp.bfloat16),
            pltpu.VMEM((2, m_chunk, n), jnp.bfloat16),
            pltpu.VMEM((m_chunk, n), jnp.float32),
            pltpu.SemaphoreType.DMA((6,)),
            pltpu.SemaphoreType.DMA((6,)),
            pltpu.SemaphoreType.REGULAR,
        ],
        compiler_params=pltpu.CompilerParams(
            collective_id=0,
            vmem_limit_bytes=40 * 1024 * 1024,
        ),
    )(x_bf, w_bf)


# device time: 189519 ns/iter; 1.6963x vs baseline; 1.6963x over previous
import jax
import jax.numpy as jnp
from jax import lax
from jax.experimental import pallas as pl
from jax.experimental.pallas import tpu as pltpu

N_DEV = 4
SUB = 2


def kernel(x, w_mat):
    m, k_shard = x.shape
    _, n = w_mat.shape
    m_chunk = m // N_DEV
    n_half = n // 2
    m_sub = m_chunk // SUB

    x_bf = x.astype(jnp.bfloat16)
    w_bf = w_mat.astype(jnp.bfloat16)

    def body(x_ref, w_ref, out_ref, send_buf, recv_buf, acc_buf,
             send_sems, recv_sems, credit_sems, out_sems):
        me = lax.axis_index("i")
        right = lax.rem(me + 1, N_DEV)
        left = lax.rem(me + N_DEV - 1, N_DEV)

        sign = (1, -1)
        send_nbr = (right, left)
        recv_nbr = (left, right)

        barrier_sem = pltpu.get_barrier_semaphore()
        for nbr in (left, right):
            pl.semaphore_signal(barrier_sem, inc=1, device_id=(nbr,),
                                device_id_type=pl.DeviceIdType.MESH)
        pl.semaphore_wait(barrier_sem, 2)

        def cmod(v):
            return lax.rem(v + 4 * N_DEV, N_DEV)

        def cols(d):
            return pl.ds(d * n_half, n_half)

        def rows_of(c, s):
            return pl.ds(c * m_chunk + s * m_sub, m_sub)

        def partial_sub(c, d, s):
            xs = x_ref[rows_of(c, s), :]
            return jnp.dot(xs, w_ref[:, cols(d)],
                           preferred_element_type=jnp.float32)

        def hop(h, d, src, dst):
            return pltpu.make_async_remote_copy(
                src_ref=src, dst_ref=dst,
                send_sem=send_sems.at[d, h], recv_sem=recv_sems.at[d, h],
                device_id=(send_nbr[d],), device_id_type=pl.DeviceIdType.MESH,
            )

        def credit(d):
            pl.semaphore_signal(credit_sems.at[d], inc=1,
                                device_id=(recv_nbr[d],),
                                device_id_type=pl.DeviceIdType.MESH)

        for d in (0, 1):
            for s in range(SUB):
                send_buf[d, 0, pl.ds(s * m_sub, m_sub), :] = (
                    partial_sub(me, d, s).astype(jnp.bfloat16))
        for h in range(N_DEV - 1):
            slot = h % 2
            rdmas = []
            for d in (0, 1):
                r = hop(h, d, send_buf.at[d, slot], recv_buf.at[d, slot])
                if h >= 2:
                    pl.semaphore_wait(credit_sems.at[d], 1)
                r.start()
                rdmas.append(r)
            for d in (0, 1):
                c = cmod(me - sign[d] * (h + 1))
                for s in range(SUB):
                    acc_buf[pl.ds(s * m_sub, m_sub), cols(d)] = (
                        partial_sub(c, d, s))
            for d in (0, 1):
                rdmas[d].wait()
                for s in range(SUB):
                    rs = pl.ds(s * m_sub, m_sub)
                    acc_buf[rs, cols(d)] = (
                        acc_buf[rs, cols(d)]
                        + recv_buf[d, slot, rs, :].astype(jnp.float32))
                if h < N_DEV - 2:
                    for s in range(SUB):
                        rs = pl.ds(s * m_sub, m_sub)
                        send_buf[d, 1 - slot, rs, :] = (
                            acc_buf[rs, cols(d)].astype(jnp.bfloat16))
                    credit(d)

        own_cp = []
        for d in (0, 1):
            own = cmod(me + sign[d])
            for s in range(SUB):
                rs = pl.ds(s * m_sub, m_sub)
                full_bf = jnp.maximum(
                    acc_buf[rs, cols(d)], 0.0).astype(jnp.bfloat16)
                send_buf[d, 1, rs, :] = full_bf
            credit(d)
            cp = pltpu.make_async_copy(
                send_buf.at[d, 1],
                out_ref.at[pl.ds(own * m_chunk, m_chunk), cols(d)],
                out_sems.at[d, 0])
            cp.start()
            own_cp.append(cp)

        r3, cp3 = [], []
        for d in (0, 1):
            r = hop(3, d, send_buf.at[d, 1], recv_buf.at[d, 1])
            pl.semaphore_wait(credit_sems.at[d], 1)
            r.start()
            r3.append(r)
        for d in (0, 1):
            r3[d].wait()
            cp = pltpu.make_async_copy(
                recv_buf.at[d, 1],
                out_ref.at[pl.ds(me * m_chunk, m_chunk), cols(d)],
                out_sems.at[d, 1])
            cp.start()
            cp3.append(cp)

        r4, cp4 = [], []
        for d in (0, 1):
            r = hop(4, d, recv_buf.at[d, 1], recv_buf.at[d, 0])
            pl.semaphore_wait(credit_sems.at[d], 1)
            r.start()
            r4.append(r)
        for d in (0, 1):
            r4[d].wait()
            cp3[d].wait()
            credit(d)
            c = cmod(me - sign[d])
            cp = pltpu.make_async_copy(
                recv_buf.at[d, 0],
                out_ref.at[pl.ds(c * m_chunk, m_chunk), cols(d)],
                out_sems.at[d, 2])
            cp.start()
            cp4.append(cp)

        r5, cp5 = [], []
        for d in (0, 1):
            r = hop(5, d, recv_buf.at[d, 0], recv_buf.at[d, 1])
            pl.semaphore_wait(credit_sems.at[d], 1)
            r.start()
            r5.append(r)
        for d in (0, 1):
            r5[d].wait()
            c = cmod(me - 2 * sign[d])
            cp = pltpu.make_async_copy(
                recv_buf.at[d, 1],
                out_ref.at[pl.ds(c * m_chunk, m_chunk), cols(d)],
                out_sems.at[d, 3])
            cp.start()
            cp5.append(cp)

        for d in (0, 1):
            own_cp[d].wait()
            cp4[d].wait()
            cp5[d].wait()

    return pl.pallas_call(
        body,
        out_shape=jax.ShapeDtypeStruct((m, n), jnp.bfloat16),
        in_specs=[pl.BlockSpec(memory_space=pltpu.VMEM),
                  pl.BlockSpec(memory_space=pltpu.VMEM)],
        out_specs=pl.BlockSpec(memory_space=pl.ANY),
        scratch_shapes=[
            pltpu.VMEM((2, 2, m_chunk, n_half), jnp.bfloat16),
            pltpu.VMEM((2, 2, m_chunk, n_half), jnp.bfloat16),
            pltpu.VMEM((m_chunk, n), jnp.float32),
            pltpu.SemaphoreType.DMA((2, 6)),
            pltpu.SemaphoreType.DMA((2, 6)),
            pltpu.SemaphoreType.REGULAR((2,)),
            pltpu.SemaphoreType.DMA((2, 4)),
        ],
        compiler_params=pltpu.CompilerParams(
            collective_id=0,
            vmem_limit_bytes=44 * 1024 * 1024,
        ),
    )(x_bf, w_bf)


# device time: 181755 ns/iter; 1.7687x vs baseline; 1.0427x over previous
import jax
import jax.numpy as jnp
from jax import lax
from jax.experimental import pallas as pl
from jax.experimental.pallas import tpu as pltpu

N_DEV = 4
SUB = 2


def kernel(x, w_mat):
    m, k_shard = x.shape
    _, n = w_mat.shape
    m_chunk = m // N_DEV
    n_half = n // 2
    m_sub = m_chunk // SUB

    x_bf = x.astype(jnp.bfloat16)
    w_bf = w_mat.astype(jnp.bfloat16)

    def body(x_ref, w_ref, out_ref, send_buf, recv_buf, part_buf,
             send_sems, recv_sems, credit_sems, out_sems):
        me = lax.axis_index("i")
        right = lax.rem(me + 1, N_DEV)
        left = lax.rem(me + N_DEV - 1, N_DEV)

        sign = (1, -1)
        send_nbr = (right, left)
        recv_nbr = (left, right)

        barrier_sem = pltpu.get_barrier_semaphore()
        for nbr in (left, right):
            pl.semaphore_signal(barrier_sem, inc=1, device_id=(nbr,),
                                device_id_type=pl.DeviceIdType.MESH)
        pl.semaphore_wait(barrier_sem, 2)

        def cmod(v):
            return lax.rem(v + 4 * N_DEV, N_DEV)

        def cols(d):
            return pl.ds(d * n_half, n_half)

        def rows_of(c, s):
            return pl.ds(c * m_chunk + s * m_sub, m_sub)

        def partial_sub(c, d, s):
            xs = x_ref[rows_of(c, s), :]
            return jnp.dot(xs, w_ref[:, cols(d)],
                           preferred_element_type=jnp.float32)

        def hop(h, d, src, dst):
            return pltpu.make_async_remote_copy(
                src_ref=src, dst_ref=dst,
                send_sem=send_sems.at[d, h], recv_sem=recv_sems.at[d, h],
                device_id=(send_nbr[d],), device_id_type=pl.DeviceIdType.MESH,
            )

        def credit(d):
            pl.semaphore_signal(credit_sems.at[d], inc=1,
                                device_id=(recv_nbr[d],),
                                device_id_type=pl.DeviceIdType.MESH)

        for d in (0, 1):
            for s in range(SUB):
                send_buf[d, 0, pl.ds(s * m_sub, m_sub), :] = (
                    partial_sub(me, d, s).astype(jnp.bfloat16))
        own_cp = []
        for h in range(N_DEV - 1):
            slot = h % 2
            last = h == N_DEV - 2
            rdmas = []
            for d in (0, 1):
                r = hop(h, d, send_buf.at[d, slot], recv_buf.at[d, slot])
                if h >= 2:
                    pl.semaphore_wait(credit_sems.at[d], 1)
                r.start()
                rdmas.append(r)
            for d in (0, 1):
                c = cmod(me - sign[d] * (h + 1))
                for s in range(SUB):
                    part_buf[d, pl.ds(s * m_sub, m_sub), :] = (
                        partial_sub(c, d, s).astype(jnp.bfloat16))
            for d in (0, 1):
                rdmas[d].wait()
                for s in range(SUB):
                    rs = pl.ds(s * m_sub, m_sub)
                    summed = part_buf[d, rs, :] + recv_buf[d, slot, rs, :]
                    if last:
                        summed = jnp.maximum(summed, 0)
                    send_buf[d, 1 - slot, rs, :] = summed
                credit(d)
            if last:
                for d in (0, 1):
                    own = cmod(me + sign[d])
                    cp = pltpu.make_async_copy(
                        send_buf.at[d, 1],
                        out_ref.at[pl.ds(own * m_chunk, m_chunk), cols(d)],
                        out_sems.at[d, 0])
                    cp.start()
                    own_cp.append(cp)

        r3, cp3 = [], []
        for d in (0, 1):
            r = hop(3, d, send_buf.at[d, 1], recv_buf.at[d, 1])
            pl.semaphore_wait(credit_sems.at[d], 1)
            r.start()
            r3.append(r)
        for d in (0, 1):
            r3[d].wait()
            cp = pltpu.make_async_copy(
                recv_buf.at[d, 1],
                out_ref.at[pl.ds(me * m_chunk, m_chunk), cols(d)],
                out_sems.at[d, 1])
            cp.start()
            cp3.append(cp)

        r4, cp4 = [], []
        for d in (0, 1):
            r = hop(4, d, recv_buf.at[d, 1], recv_buf.at[d, 0])
            pl.semaphore_wait(credit_sems.at[d], 1)
            r.start()
            r4.append(r)
        for d in (0, 1):
            r4[d].wait()
            cp3[d].wait()
            credit(d)
            c = cmod(me - sign[d])
            cp = pltpu.make_async_copy(
                recv_buf.at[d, 0],
                out_ref.at[pl.ds(c * m_chunk, m_chunk), cols(d)],
                out_sems.at[d, 2])
            cp.start()
            cp4.append(cp)

        r5, cp5 = [], []
        for d in (0, 1):
            r = hop(5, d, recv_buf.at[d, 0], recv_buf.at[d, 1])
            pl.semaphore_wait(credit_sems.at[d], 1)
            r.start()
            r5.append(r)
        for d in (0, 1):
            r5[d].wait()
            c = cmod(me - 2 * sign[d])
            cp = pltpu.make_async_copy(
                recv_buf.at[d, 1],
                out_ref.at[pl.ds(c * m_chunk, m_chunk), cols(d)],
                out_sems.at[d, 3])
            cp.start()
            cp5.append(cp)

        for d in (0, 1):
            own_cp[d].wait()
            cp4[d].wait()
            cp5[d].wait()

    return pl.pallas_call(
        body,
        out_shape=jax.ShapeDtypeStruct((m, n), jnp.bfloat16),
        in_specs=[pl.BlockSpec(memory_space=pltpu.VMEM),
                  pl.BlockSpec(memory_space=pltpu.VMEM)],
        out_specs=pl.BlockSpec(memory_space=pl.ANY),
        scratch_shapes=[
            pltpu.VMEM((2, 2, m_chunk, n_half), jnp.bfloat16),
            pltpu.VMEM((2, 2, m_chunk, n_half), jnp.bfloat16),
            pltpu.VMEM((2, m_chunk, n_half), jnp.bfloat16),
            pltpu.SemaphoreType.DMA((2, 6)),
            pltpu.SemaphoreType.DMA((2, 6)),
            pltpu.SemaphoreType.REGULAR((2,)),
            pltpu.SemaphoreType.DMA((2, 4)),
        ],
        compiler_params=pltpu.CompilerParams(
            collective_id=0,
            vmem_limit_bytes=40 * 1024 * 1024,
        ),
    )(x_bf, w_bf)


# device time: 161158 ns/iter; 1.9948x vs baseline; 1.1278x over previous
import jax
import jax.numpy as jnp
from jax import lax
from jax.experimental import pallas as pl
from jax.experimental.pallas import tpu as pltpu

N_DEV = 4
N_HOP = 2 * (N_DEV - 1)
Q = 4


def kernel(x, w_mat):
    m, k_shard = x.shape
    _, n = w_mat.shape
    m_chunk = m // N_DEV
    n_half = n // 2
    m_q = m_chunk // Q

    def body(x_ref, w_ref, out_ref, send_buf, recv_buf, x_bf, w_bf,
             xstage, send_sems, recv_sems, credit_sems, out_sems,
             xdma_sems):
        me = lax.axis_index("i")
        right = lax.rem(me + 1, N_DEV)
        left = lax.rem(me + N_DEV - 1, N_DEV)

        sign = (1, -1)
        send_nbr = (right, left)
        recv_nbr = (left, right)

        def cmod(v):
            return lax.rem(v + 4 * N_DEV, N_DEV)

        def cols(d):
            return pl.ds(d * n_half, n_half)

        def qrows(c, q):
            return pl.ds(c * m_chunk + q * m_q, m_q)

        def fetch(c, st, i):
            return pltpu.make_async_copy(
                x_ref.at[pl.ds(c * m_chunk, m_chunk), :],
                xstage.at[st], xdma_sems.at[i])

        f0 = fetch(me, 0, 0)
        f0.start()
        f1 = fetch(cmod(me - 1), 1, 1)
        f1.start()

        barrier_sem = pltpu.get_barrier_semaphore()
        for nbr in (left, right):
            pl.semaphore_signal(barrier_sem, inc=1, device_id=(nbr,),
                                device_id_type=pl.DeviceIdType.MESH)
        pl.semaphore_wait(barrier_sem, 2)

        def cast_q(c, st, q):
            x_bf[qrows(c, q), :] = (
                xstage[st, pl.ds(q * m_q, m_q), :].astype(jnp.bfloat16))

        def dot_q(c, d, q):
            return jnp.dot(x_bf[qrows(c, q), :], w_bf[:, cols(d)],
                           preferred_element_type=jnp.float32
                           ).astype(jnp.bfloat16)

        def credit(d, q):
            pl.semaphore_signal(credit_sems.at[d, q], inc=1,
                                device_id=(recv_nbr[d],),
                                device_id_type=pl.DeviceIdType.MESH)

        def hop_bufs(h, d, q):
            if h < N_DEV - 1:
                return send_buf.at[d, h % 2, q], recv_buf.at[d, h % 2, q]
            if h == 3:
                return send_buf.at[d, 1, q], recv_buf.at[d, 1, q]
            if h == 4:
                return recv_buf.at[d, 1, q], recv_buf.at[d, 0, q]
            return (recv_buf.at[d, 0, q],
                    out_ref.at[qrows(cmod(me - sign[d]), q), cols(d)])

        rdmas = {}
        out_cps = {}

        def start_hop(q, h):
            rs = []
            for d in (0, 1):
                src, dst = hop_bufs(h, d, q)
                r = pltpu.make_async_remote_copy(
                    src_ref=src, dst_ref=dst,
                    send_sem=send_sems.at[d, h, q],
                    recv_sem=recv_sems.at[d, h, q],
                    device_id=(send_nbr[d],),
                    device_id_type=pl.DeviceIdType.MESH)
                if 2 <= h <= 4:
                    pl.semaphore_wait(credit_sems.at[d, q], 1)
                r.start()
                rs.append(r)
            rdmas[(q, h)] = rs

        def out_copy(q, j, src, rows):
            cps = []
            for d in (0, 1):
                cp = pltpu.make_async_copy(
                    src(d), out_ref.at[rows(d), cols(d)],
                    out_sems.at[d, j, q])
                cp.start()
                cps.append(cp)
            out_cps[(q, j)] = cps

        def finish_hop(q, h):
            slot = h % 2
            for d in (0, 1):
                rdmas[(q, h)][d].wait()
            if h < N_DEV - 1:
                for d in (0, 1):
                    summed = (send_buf[d, 1 - slot, q, :, :]
                              + recv_buf[d, slot, q, :, :])
                    if h == N_DEV - 2:
                        summed = jnp.maximum(summed, 0)
                    send_buf[d, 1 - slot, q, :, :] = summed
                    credit(d, q)
                if h == N_DEV - 2:
                    out_copy(q, 0, lambda d: send_buf.at[d, 1, q],
                             lambda d: qrows(cmod(me + sign[d]), q))
            elif h == 3:
                out_copy(q, 1, lambda d: recv_buf.at[d, 1, q],
                         lambda d: qrows(me, q))
            elif h == 4:
                out_copy(q, 2, lambda d: recv_buf.at[d, 0, q],
                         lambda d: qrows(cmod(me - sign[d]), q))

        for q in range(Q):
            if q == 0:
                w_bf[:, cols(0)] = w_ref[:, cols(0)].astype(jnp.bfloat16)
                f0.wait()
            cast_q(me, 0, q)
            pair = []
            for d in (0, 1):
                if q == 0 and d == 1:
                    w_bf[:, cols(1)] = (
                        w_ref[:, cols(1)].astype(jnp.bfloat16))
                send_buf[d, 0, q, :, :] = dot_q(me, d, q)
                src, dst = hop_bufs(0, d, q)
                r = pltpu.make_async_remote_copy(
                    src_ref=src, dst_ref=dst,
                    send_sem=send_sems.at[d, 0, q],
                    recv_sem=recv_sems.at[d, 0, q],
                    device_id=(send_nbr[d],),
                    device_id_type=pl.DeviceIdType.MESH)
                r.start()
                pair.append(r)
            rdmas[(q, 0)] = pair

        f2 = fetch(cmod(me + 1), 0, 2)
        f2.start()
        for h in range(N_HOP):
            nxt = h + 1
            for q in range(Q):
                if nxt <= N_DEV - 1:
                    chunks = []
                    if nxt == 1:
                        if q == 0:
                            f1.wait()
                            f2.wait()
                        chunks = [(cmod(me - 1), 1), (cmod(me + 1), 0)]
                    elif nxt == 2:
                        if q == 0:
                            f3.wait()
                        chunks = [(cmod(me + 2), 1)]
                    for c, st in chunks:
                        cast_q(c, st, q)
                    if nxt == 1 and q == Q - 1:
                        f3 = fetch(cmod(me + 2), 1, 3)
                        f3.start()
                    for d in (0, 1):
                        send_buf[d, nxt % 2, q, :, :] = (
                            dot_q(cmod(me - sign[d] * nxt), d, q))
                finish_hop(q, h)
                if nxt < N_HOP:
                    start_hop(q, nxt)

        for q in range(Q):
            for j in (0, 1, 2):
                for d in (0, 1):
                    out_cps[(q, j)][d].wait()

    return pl.pallas_call(
        body,
        out_shape=jax.ShapeDtypeStruct((m, n), jnp.bfloat16),
        in_specs=[pl.BlockSpec(memory_space=pl.ANY),
                  pl.BlockSpec(memory_space=pltpu.VMEM)],
        out_specs=pl.BlockSpec(memory_space=pl.ANY),
        scratch_shapes=[
            pltpu.VMEM((2, 2, Q, m_q, n_half), jnp.bfloat16),
            pltpu.VMEM((2, 2, Q, m_q, n_half), jnp.bfloat16),
            pltpu.VMEM((m, k_shard), jnp.bfloat16),
            pltpu.VMEM((k_shard, n), jnp.bfloat16),
            pltpu.VMEM((2, m_chunk, k_shard), jnp.float32),
            pltpu.SemaphoreType.DMA((2, N_HOP, Q)),
            pltpu.SemaphoreType.DMA((2, N_HOP, Q)),
            pltpu.SemaphoreType.REGULAR((2, Q)),
            pltpu.SemaphoreType.DMA((2, 3, Q)),
            pltpu.SemaphoreType.DMA((4,)),
        ],
        compiler_params=pltpu.CompilerParams(
            collective_id=0,
            vmem_limit_bytes=54 * 1024 * 1024,
        ),
    )(x, w_mat)
